# baseline (speedup 1.0000x reference)
import functools

import jax
import jax.numpy as jnp
from jax import lax
from jax.experimental import pallas as pl
from jax.experimental.pallas import tpu as pltpu
from jax.experimental.pallas import tpu_sc as plsc

_NC = 2
_NS = 16
_NW = _NC * _NS
_L = 16


def _transpose_tile_to(out_buf, in_buf, ncols):
    lane = lax.iota(jnp.int32, _L)
    for l in range(ncols):
        for d0 in (0, 16):
            vals = plsc.load_gather(
                in_buf, [d0 + lane, jnp.full((_L,), l, jnp.int32)]
            )
            flat = 32 * l + d0
            out_buf[flat // 1024, (flat // 128) % 8, pl.ds(flat % 128, _L)] = vals


@functools.lru_cache(maxsize=None)
def _make_transpose(V, D):
    assert D == 32
    ntc = (V + 127) // 128
    iters = ntc // _NW + 1
    mesh = plsc.VectorSubcoreMesh(core_axis_name="c", subcore_axis_name="s")

    @functools.partial(
        pl.kernel,
        mesh=mesh,
        compiler_params=pltpu.CompilerParams(needs_layout_passes=False),
        out_type=jax.ShapeDtypeStruct((4 * ntc, 8, 128), jnp.float32),
        scratch_types=[
            pltpu.VMEM((2, D, 128), jnp.float32),
            pltpu.VMEM((2, 4, 8, 128), jnp.float32),
            pltpu.SemaphoreType.DMA,
            pltpu.SemaphoreType.DMA,
            pltpu.SemaphoreType.DMA,
            pltpu.SemaphoreType.DMA,
        ],
    )
    def transpose_kernel(wt, out, in_v, tr_v, sem_a, sem_b, sem_oa, sem_ob):
        w = lax.axis_index("s") * _NC + lax.axis_index("c")
        sems = (sem_a, sem_b)
        osems = (sem_oa, sem_ob)

        def col_of(t):
            return t * _NW + w

        def start_read(t, h):
            c = col_of(t)

            @pl.when(c < ntc)
            def _():
                pltpu.async_copy(
                    wt.at[:, pl.ds(pl.multiple_of(c * 128, 128), 128)],
                    in_v.at[h],
                    sems[h],
                )

        start_read(0, 0)

        def body(t, _):
            for h in (0, 1):
                tt = 2 * t + h
                c = col_of(tt)
                start_read(tt + 1, 1 - h)

                @pl.when(c < ntc)
                def _():
                    pltpu.make_async_copy(
                        wt.at[:, pl.ds(0, 128)], in_v.at[h], sems[h]
                    ).wait()

                    @pl.when(tt >= 2)
                    def _():
                        pltpu.make_async_copy(
                            tr_v.at[h], out.at[pl.ds(0, 4)], osems[h]
                        ).wait()

                    _transpose_tile_to(tr_v.at[h], in_v.at[h], 128)
                    pltpu.async_copy(tr_v.at[h], out.at[pl.ds(4 * c, 4)], osems[h])

            return 0

        lax.fori_loop(0, (iters + 1) // 2, body, 0)

        for h in (0, 1):
            pltpu.make_async_copy(tr_v.at[h], out.at[pl.ds(0, 4)], osems[h]).wait()

    return transpose_kernel


@functools.lru_cache(maxsize=None)
def _make_gather(V, D, J, B):
    assert B % (_NW * 128) == 0 and D == 32
    ntc = B // 128
    mesh = plsc.VectorSubcoreMesh(core_axis_name="c", subcore_axis_name="s")

    @functools.partial(
        pl.kernel,
        mesh=mesh,
        compiler_params=pltpu.CompilerParams(
            use_tc_tiling_on_sc=False, needs_layout_passes=False
        ),
        out_type=jax.ShapeDtypeStruct((J, D // 8, ntc, 8, 128), jnp.float32),
        scratch_types=[
            pltpu.VMEM((J, 128), jnp.int32),
            pltpu.VMEM((2, 128, D), jnp.float32),
            pltpu.VMEM((2, 4, 8, 128), jnp.float32),
            pltpu.SemaphoreType.DMA,
            pltpu.SemaphoreType.DMA,
            pltpu.SemaphoreType.DMA,
            pltpu.SemaphoreType.DMA,
            pltpu.SemaphoreType.DMA,
        ],
    )
    def gather_kernel(
        table, idx, out, idx_v, rows_v, perm_v, sem_i, sem_a, sem_b, sem_oa, sem_ob
    ):
        w = lax.axis_index("s") * _NC + lax.axis_index("c")
        tc = w
        lane = lax.iota(jnp.int32, _L)
        sems = (sem_a, sem_b)
        osems = (sem_oa, sem_ob)

        pltpu.async_copy(idx.at[:, pl.ds(tc * 128, 128)], idx_v, sem_i).wait()

        def start_gather(j, h):
            @pl.when(j < J)
            def _():
                pltpu.async_copy(table.at[idx_v.at[j]], rows_v.at[h], sems[h])

        start_gather(0, 0)

        def body(jj, _):
            for h in (0, 1):
                j = 2 * jj + h
                start_gather(j + 1, 1 - h)
                pltpu.make_async_copy(
                    table.at[pl.ds(0, 128)], rows_v.at[h], sems[h]
                ).wait()

                @pl.when(j >= 2)
                def _():
                    pltpu.make_async_copy(
                        perm_v.at[h], out.at[0, :, 0], osems[h]
                    ).wait()

                for d in range(D):
                    for l0 in range(0, 128, _L):
                        vals = plsc.load_gather(
                            rows_v.at[h],
                            [l0 + lane, jnp.full((_L,), d, jnp.int32)],
                        )
                        perm_v[h, d // 8, d % 8, pl.ds(l0, _L)] = vals
                pltpu.async_copy(perm_v.at[h], out.at[j, :, tc], osems[h])

            return 0

        lax.fori_loop(0, J // 2, body, 0)

        for h in (0, 1):
            pltpu.make_async_copy(perm_v.at[h], out.at[0, :, 0], osems[h]).wait()

    return gather_kernel


def kernel(x, weight):
    Bq, J = x.shape
    V, D = weight.shape
    wt_lin = _make_transpose(V, D)(weight.T)
    V_pad = wt_lin.shape[0] * 1024 // D
    table = wt_lin.reshape(V_pad, D)
    idx = x.T.astype(jnp.int32)
    out5 = _make_gather(V, D, J, Bq)(table, idx)
    return out5.transpose(2, 4, 0, 1, 3).reshape(Bq, J, D)


# speedup vs baseline: 1.8774x; 1.8774x over previous
import functools

import jax
import jax.numpy as jnp
from jax import lax
from jax.experimental import pallas as pl
from jax.experimental.pallas import tpu as pltpu
from jax.experimental.pallas import tpu_sc as plsc

_NC = 2
_NS = 16
_NW = _NC * _NS
_L = 16


def _transpose_tile_to(out_buf, in_buf, ncols):
    lane = lax.iota(jnp.int32, _L)

    @plsc.parallel_loop(0, ncols, unroll=4)
    def _(l):
        for d0 in (0, 16):
            vals = plsc.load_gather(
                in_buf, [d0 + lane, jnp.full((_L,), l, jnp.int32)]
            )
            flat = 32 * l + d0
            out_buf[flat // 1024, (flat // 128) % 8, pl.ds(flat % 128, _L)] = vals


@functools.lru_cache(maxsize=None)
def _make_transpose(V, D):
    assert D == 32
    ntc = (V + 127) // 128
    iters = ntc // _NW + 1
    mesh = plsc.VectorSubcoreMesh(core_axis_name="c", subcore_axis_name="s")

    @functools.partial(
        pl.kernel,
        mesh=mesh,
        compiler_params=pltpu.CompilerParams(needs_layout_passes=False),
        out_type=jax.ShapeDtypeStruct((4 * ntc, 8, 128), jnp.float32),
        scratch_types=[
            pltpu.VMEM((2, D, 128), jnp.float32),
            pltpu.VMEM((2, 4, 8, 128), jnp.float32),
            pltpu.SemaphoreType.DMA,
            pltpu.SemaphoreType.DMA,
            pltpu.SemaphoreType.DMA,
            pltpu.SemaphoreType.DMA,
        ],
    )
    def transpose_kernel(wt, out, in_v, tr_v, sem_a, sem_b, sem_oa, sem_ob):
        w = lax.axis_index("s") * _NC + lax.axis_index("c")
        sems = (sem_a, sem_b)
        osems = (sem_oa, sem_ob)

        def col_of(t):
            return t * _NW + w

        def start_read(t, h):
            c = col_of(t)

            @pl.when(c < ntc)
            def _():
                pltpu.async_copy(
                    wt.at[:, pl.ds(pl.multiple_of(c * 128, 128), 128)],
                    in_v.at[h],
                    sems[h],
                )

        start_read(0, 0)

        def body(t, _):
            for h in (0, 1):
                tt = 2 * t + h
                c = col_of(tt)
                start_read(tt + 1, 1 - h)

                @pl.when(c < ntc)
                def _():
                    pltpu.make_async_copy(
                        wt.at[:, pl.ds(0, 128)], in_v.at[h], sems[h]
                    ).wait()

                    @pl.when(tt >= 2)
                    def _():
                        pltpu.make_async_copy(
                            tr_v.at[h], out.at[pl.ds(0, 4)], osems[h]
                        ).wait()

                    _transpose_tile_to(tr_v.at[h], in_v.at[h], 128)
                    pltpu.async_copy(tr_v.at[h], out.at[pl.ds(4 * c, 4)], osems[h])

            return 0

        lax.fori_loop(0, (iters + 1) // 2, body, 0)

        for h in (0, 1):
            pltpu.make_async_copy(tr_v.at[h], out.at[pl.ds(0, 4)], osems[h]).wait()

    return transpose_kernel


@functools.lru_cache(maxsize=None)
def _make_gather(V, D, J, B):
    assert B % (_NW * 128) == 0 and D == 32
    ntc = B // 128
    mesh = plsc.VectorSubcoreMesh(core_axis_name="c", subcore_axis_name="s")

    @functools.partial(
        pl.kernel,
        mesh=mesh,
        compiler_params=pltpu.CompilerParams(
            use_tc_tiling_on_sc=False, needs_layout_passes=False
        ),
        out_type=jax.ShapeDtypeStruct((J, D // 8, ntc, 8, 128), jnp.float32),
        scratch_types=[
            pltpu.VMEM((J, 128), jnp.int32),
            pltpu.VMEM((2, 128, D), jnp.float32),
            pltpu.VMEM((2, 4, 8, 128), jnp.float32),
            pltpu.SemaphoreType.DMA,
            pltpu.SemaphoreType.DMA,
            pltpu.SemaphoreType.DMA,
            pltpu.SemaphoreType.DMA,
            pltpu.SemaphoreType.DMA,
        ],
    )
    def gather_kernel(
        table, idx, out, idx_v, rows_v, perm_v, sem_i, sem_a, sem_b, sem_oa, sem_ob
    ):
        w = lax.axis_index("s") * _NC + lax.axis_index("c")
        tc = w
        lane = lax.iota(jnp.int32, _L)
        sems = (sem_a, sem_b)
        osems = (sem_oa, sem_ob)

        pltpu.async_copy(idx.at[:, pl.ds(tc * 128, 128)], idx_v, sem_i).wait()

        def start_gather(j, h):
            @pl.when(j < J)
            def _():
                pltpu.async_copy(table.at[idx_v.at[j]], rows_v.at[h], sems[h])

        start_gather(0, 0)

        def body(jj, _):
            for h in (0, 1):
                j = 2 * jj + h
                start_gather(j + 1, 1 - h)
                pltpu.make_async_copy(
                    table.at[pl.ds(0, 128)], rows_v.at[h], sems[h]
                ).wait()

                @pl.when(j >= 2)
                def _():
                    pltpu.make_async_copy(
                        perm_v.at[h], out.at[0, :, 0], osems[h]
                    ).wait()

                @plsc.parallel_loop(0, D, unroll=2)
                def _(d):
                    for l0 in range(0, 128, _L):
                        vals = plsc.load_gather(
                            rows_v.at[h],
                            [l0 + lane, jnp.full((_L,), d, jnp.int32)],
                        )
                        perm_v[h, d // 8, d % 8, pl.ds(l0, _L)] = vals

                pltpu.async_copy(perm_v.at[h], out.at[j, :, tc], osems[h])

            return 0

        lax.fori_loop(0, J // 2, body, 0)

        for h in (0, 1):
            pltpu.make_async_copy(perm_v.at[h], out.at[0, :, 0], osems[h]).wait()

    return gather_kernel


def kernel(x, weight):
    Bq, J = x.shape
    V, D = weight.shape
    wt_lin = _make_transpose(V, D)(weight.T)
    V_pad = wt_lin.shape[0] * 1024 // D
    table = wt_lin.reshape(V_pad, D)
    idx = x.T.astype(jnp.int32)
    out5 = _make_gather(V, D, J, Bq)(table, idx)
    return out5.transpose(2, 4, 0, 1, 3).reshape(Bq, J, D)
